# Initial kernel scaffold; baseline (speedup 1.0000x reference)
import jax
import jax.numpy as jnp
from jax.experimental import pallas as pl


def kernel(poi_input, time_input, poi_table, time_table, W, b):
    raise NotImplementedError("write your pallas kernel here")


# speedup vs baseline: 2.5472x; 2.5472x over previous
import functools

import jax
import jax.numpy as jnp
from jax import lax
from jax.experimental import pallas as pl
from jax.experimental.pallas import tpu as pltpu
from jax.experimental.pallas import tpu_sc as plsc

EMBED = 128
TIME_D = 32
TIME_V = 24
_ROWS_PER_BLOCK = 2000
_TIME_BLOCK = 2048


def _transform_body(x_ref, w_ref, b_ref, o_ref):
    o_ref[...] = lax.dot_general(
        x_ref[...], w_ref[...],
        dimension_numbers=(((1,), (1,)), ((), ())),
        preferred_element_type=jnp.float32,
    ) + b_ref[...]


def _transform_table(table, W, b):
    n = table.shape[0]
    grid = n // _ROWS_PER_BLOCK
    return pl.pallas_call(
        _transform_body,
        grid=(grid,),
        in_specs=[
            pl.BlockSpec((_ROWS_PER_BLOCK, EMBED), lambda i: (i, 0)),
            pl.BlockSpec((EMBED, EMBED), lambda i: (0, 0)),
            pl.BlockSpec((1, EMBED), lambda i: (0, 0)),
        ],
        out_specs=pl.BlockSpec((_ROWS_PER_BLOCK, EMBED), lambda i: (i, 0)),
        out_shape=jax.ShapeDtypeStruct((n, EMBED), jnp.float32),
    )(table, W, b.reshape(1, EMBED))


def _time_body(idx_ref, tab_ref, o_ref):
    idx = idx_ref[...]
    iot = lax.broadcasted_iota(jnp.int32, (_TIME_BLOCK, TIME_V), 1)
    onehot = (idx == iot).astype(jnp.float32)
    o_ref[...] = lax.dot_general(
        onehot, tab_ref[...],
        dimension_numbers=(((1,), (0,)), ((), ())),
        preferred_element_type=jnp.float32,
    )


def _time_lookup(time_idx_flat, time_table):
    n = time_idx_flat.shape[0]
    grid = n // _TIME_BLOCK
    return pl.pallas_call(
        _time_body,
        grid=(grid,),
        in_specs=[
            pl.BlockSpec((_TIME_BLOCK, 1), lambda i: (i, 0)),
            pl.BlockSpec((TIME_V, TIME_D), lambda i: (0, 0)),
        ],
        out_specs=pl.BlockSpec((_TIME_BLOCK, TIME_D), lambda i: (i, 0)),
        out_shape=jax.ShapeDtypeStruct((n, TIME_D), jnp.float32),
    )(time_idx_flat.reshape(n, 1), time_table)


@functools.lru_cache(maxsize=None)
def _make_sc_gather(B):
    info = plsc.get_sparse_core_info()
    NC, NS = info.num_cores, info.num_subcores
    NW = NC * NS
    rows_per_w = B // NW
    CH = 128
    n_ch = rows_per_w // CH
    mesh = plsc.VectorSubcoreMesh(core_axis_name="c", subcore_axis_name="s")

    @functools.partial(
        pl.kernel,
        mesh=mesh,
        out_type=jax.ShapeDtypeStruct((B, EMBED), jnp.float32),
        scratch_types=[
            pltpu.VMEM((n_ch, CH), jnp.int32),
            pltpu.VMEM((CH, EMBED), jnp.float32),
            pltpu.SemaphoreType.DMA,
        ],
    )
    def sc_gather(ptab_hbm, pidx_hbm, pout_hbm, pidx_v, prow_v, psem):
        wid = lax.axis_index("s") * NC + lax.axis_index("c")
        pltpu.sync_copy(pidx_hbm.at[wid], pidx_v)

        def body(j, carry):
            out0 = wid * rows_per_w + j * CH
            cp = pltpu.async_copy(ptab_hbm.at[pidx_v.at[j]], prow_v, psem)
            cp.wait()
            pltpu.sync_copy(prow_v, pout_hbm.at[pl.ds(out0, CH)])
            return carry

        lax.fori_loop(0, n_ch, body, 0)

    return sc_gather


def kernel(poi_input, time_input, poi_table, time_table, W, b):
    bsz, hist = poi_input.shape
    B = bsz * hist
    t_table = _transform_table(poi_table, W, b)
    time_out = _time_lookup(time_input.reshape(B), time_table)
    info = plsc.get_sparse_core_info()
    nw = info.num_cores * info.num_subcores
    pidx = poi_input.reshape(nw, B // (nw * 128), 128)
    poi_out = _make_sc_gather(B)(t_table, pidx)
    return (poi_out.reshape(bsz, hist, EMBED),
            time_out.reshape(bsz, hist, TIME_D))


# speedup vs baseline: 9.2643x; 3.6371x over previous
import functools

import jax
import jax.numpy as jnp
from jax import lax
from jax.experimental import pallas as pl
from jax.experimental.pallas import tpu as pltpu
from jax.experimental.pallas import tpu_sc as plsc

EMBED = 128
TIME_D = 32
TIME_V = 24
_ROWS_PER_BLOCK = 2000
_TIME_BLOCK = 2048


def _transform_body(x_ref, w_ref, b_ref, o_ref):
    o_ref[...] = lax.dot_general(
        x_ref[...], w_ref[...],
        dimension_numbers=(((1,), (1,)), ((), ())),
        preferred_element_type=jnp.float32,
    ) + b_ref[...]


def _transform_table(table, W, b):
    n = table.shape[0]
    grid = n // _ROWS_PER_BLOCK
    return pl.pallas_call(
        _transform_body,
        grid=(grid,),
        in_specs=[
            pl.BlockSpec((_ROWS_PER_BLOCK, EMBED), lambda i: (i, 0)),
            pl.BlockSpec((EMBED, EMBED), lambda i: (0, 0)),
            pl.BlockSpec((1, EMBED), lambda i: (0, 0)),
        ],
        out_specs=pl.BlockSpec((_ROWS_PER_BLOCK, EMBED), lambda i: (i, 0)),
        out_shape=jax.ShapeDtypeStruct((n, EMBED), jnp.float32),
    )(table, W, b.reshape(1, EMBED))


def _time_body(idx_ref, tab_ref, o_ref):
    bsz = idx_ref.shape[-1]
    idx = idx_ref[0]
    iot = lax.broadcasted_iota(jnp.int32, (TIME_V, bsz), 0)
    onehot = (jnp.broadcast_to(idx, (TIME_V, bsz)) == iot).astype(jnp.float32)
    o_ref[0] = lax.dot_general(
        tab_ref[...], onehot,
        dimension_numbers=(((0,), (0,)), ((), ())),
        preferred_element_type=jnp.float32,
    )


def _time_lookup(time_idx_t, time_table):
    hist, bsz = time_idx_t.shape
    return pl.pallas_call(
        _time_body,
        grid=(hist,),
        in_specs=[
            pl.BlockSpec((1, 1, bsz), lambda i: (i, 0, 0)),
            pl.BlockSpec((TIME_V, TIME_D), lambda i: (0, 0)),
        ],
        out_specs=pl.BlockSpec((1, TIME_D, bsz), lambda i: (i, 0, 0)),
        out_shape=jax.ShapeDtypeStruct((hist, TIME_D, bsz), jnp.float32),
    )(time_idx_t.reshape(hist, 1, bsz), time_table)


@functools.lru_cache(maxsize=None)
def _make_sc_gather(B):
    info = plsc.get_sparse_core_info()
    NC, NS = info.num_cores, info.num_subcores
    NW = NC * NS
    rows_per_w = B // NW
    CH = 128
    n_ch = rows_per_w // CH
    mesh = plsc.VectorSubcoreMesh(core_axis_name="c", subcore_axis_name="s")

    @functools.partial(
        pl.kernel,
        mesh=mesh,
        out_type=jax.ShapeDtypeStruct((B, EMBED), jnp.float32),
        scratch_types=[
            pltpu.VMEM((n_ch, CH), jnp.int32),
            pltpu.VMEM((CH, EMBED), jnp.float32),
            pltpu.SemaphoreType.DMA,
        ],
        compiler_params=pltpu.CompilerParams(use_tc_tiling_on_sc=True),
    )
    def sc_gather(ptab_hbm, pidx_hbm, pout_hbm, pidx_v, prow_v, psem):
        wid = lax.axis_index("s") * NC + lax.axis_index("c")
        pltpu.sync_copy(pidx_hbm.at[wid], pidx_v)

        def body(j, carry):
            out0 = wid * rows_per_w + j * CH
            cp = pltpu.async_copy(ptab_hbm.at[pidx_v.at[j]], prow_v, psem)
            cp.wait()
            pltpu.sync_copy(prow_v, pout_hbm.at[pl.ds(out0, CH)])
            return carry

        lax.fori_loop(0, n_ch, body, 0)

    return sc_gather


def kernel(poi_input, time_input, poi_table, time_table, W, b):
    bsz, hist = poi_input.shape
    B = bsz * hist
    t_table = _transform_table(poi_table, W, b)
    time_out = _time_lookup(time_input.T, time_table)
    info = plsc.get_sparse_core_info()
    nw = info.num_cores * info.num_subcores
    pidx = poi_input.T.reshape(nw, B // (nw * 128), 128)
    poi_out = _make_sc_gather(B)(t_table, pidx)
    return (poi_out.reshape(hist, bsz, EMBED).transpose(1, 0, 2),
            time_out.transpose(2, 0, 1))


# speedup vs baseline: 11.0032x; 1.1877x over previous
import functools

import jax
import jax.numpy as jnp
from jax import lax
from jax.experimental import pallas as pl
from jax.experimental.pallas import tpu as pltpu
from jax.experimental.pallas import tpu_sc as plsc

EMBED = 128
TIME_D = 32
TIME_V = 24
_ROWS_PER_BLOCK = 2000
_TIME_BLOCK = 2048


def _transform_body(x_ref, w_ref, b_ref, o_ref):
    o_ref[...] = lax.dot_general(
        x_ref[...], w_ref[...],
        dimension_numbers=(((1,), (1,)), ((), ())),
        preferred_element_type=jnp.float32,
    ) + b_ref[...]


def _transform_table(table, W, b):
    n = table.shape[0]
    grid = n // _ROWS_PER_BLOCK
    return pl.pallas_call(
        _transform_body,
        grid=(grid,),
        in_specs=[
            pl.BlockSpec((_ROWS_PER_BLOCK, EMBED), lambda i: (i, 0)),
            pl.BlockSpec((EMBED, EMBED), lambda i: (0, 0)),
            pl.BlockSpec((1, EMBED), lambda i: (0, 0)),
        ],
        out_specs=pl.BlockSpec((_ROWS_PER_BLOCK, EMBED), lambda i: (i, 0)),
        out_shape=jax.ShapeDtypeStruct((n, EMBED), jnp.float32),
    )(table, W, b.reshape(1, EMBED))


def _time_body(idx_ref, tab_ref, o_ref):
    bsz = idx_ref.shape[-1]
    idx = idx_ref[0]
    iot = lax.broadcasted_iota(jnp.int32, (TIME_V, bsz), 0)
    onehot = (jnp.broadcast_to(idx, (TIME_V, bsz)) == iot).astype(jnp.float32)
    o_ref[0] = lax.dot_general(
        tab_ref[...], onehot,
        dimension_numbers=(((0,), (0,)), ((), ())),
        preferred_element_type=jnp.float32,
    )


def _time_lookup(time_idx_t, time_table):
    hist, bsz = time_idx_t.shape
    return pl.pallas_call(
        _time_body,
        grid=(hist,),
        in_specs=[
            pl.BlockSpec((1, 1, bsz), lambda i: (i, 0, 0)),
            pl.BlockSpec((TIME_V, TIME_D), lambda i: (0, 0)),
        ],
        out_specs=pl.BlockSpec((1, TIME_D, bsz), lambda i: (i, 0, 0)),
        out_shape=jax.ShapeDtypeStruct((hist, TIME_D, bsz), jnp.float32),
    )(time_idx_t.reshape(hist, 1, bsz), time_table)


@functools.lru_cache(maxsize=None)
def _make_sc_gather(B):
    info = plsc.get_sparse_core_info()
    NC, NS = info.num_cores, info.num_subcores
    NW = NC * NS
    rows_per_w = B // NW
    CH = 128
    n_ch = rows_per_w // CH
    NBUF = 5
    n_grp = n_ch // NBUF
    mesh = plsc.VectorSubcoreMesh(core_axis_name="c", subcore_axis_name="s")

    @functools.partial(
        pl.kernel,
        mesh=mesh,
        out_type=jax.ShapeDtypeStruct((B, EMBED), jnp.float32),
        scratch_types=[
            pltpu.VMEM((n_ch, CH), jnp.int32),
            pltpu.VMEM((NBUF, CH, EMBED), jnp.float32),
            pltpu.SemaphoreType.DMA,
            pltpu.SemaphoreType.DMA,
        ],
        compiler_params=pltpu.CompilerParams(use_tc_tiling_on_sc=True),
    )
    def sc_gather(ptab_hbm, pidx_hbm, pout_hbm, pidx_v, prow_v, psem, osem):
        wid = lax.axis_index("s") * NC + lax.axis_index("c")
        pltpu.sync_copy(pidx_hbm.at[wid], pidx_v)

        def body(g, carry):
            j0 = g * NBUF
            cps = [pltpu.async_copy(ptab_hbm.at[pidx_v.at[j0 + k]],
                                    prow_v.at[k], psem)
                   for k in range(NBUF)]
            ocs = []
            for k in range(NBUF):
                cps[k].wait()
                out0 = wid * rows_per_w + (j0 + k) * CH
                ocs.append(pltpu.async_copy(
                    prow_v.at[k], pout_hbm.at[pl.ds(out0, CH)], osem))
            for oc in ocs:
                oc.wait()
            return carry

        lax.fori_loop(0, n_grp, body, 0)

    return sc_gather


def kernel(poi_input, time_input, poi_table, time_table, W, b):
    bsz, hist = poi_input.shape
    B = bsz * hist
    t_table = _transform_table(poi_table, W, b)
    time_out = _time_lookup(time_input.T, time_table)
    info = plsc.get_sparse_core_info()
    nw = info.num_cores * info.num_subcores
    pidx = poi_input.T.reshape(nw, B // (nw * 128), 128)
    poi_out = _make_sc_gather(B)(t_table, pidx)
    return (poi_out.reshape(hist, bsz, EMBED).transpose(1, 0, 2),
            time_out.transpose(2, 0, 1))


# speedup vs baseline: 12.3679x; 1.1240x over previous
import functools

import jax
import jax.numpy as jnp
from jax import lax
from jax.experimental import pallas as pl
from jax.experimental.pallas import tpu as pltpu
from jax.experimental.pallas import tpu_sc as plsc

EMBED = 128
TIME_D = 32
TIME_V = 24
_ROWS_PER_BLOCK = 4000
_TIME_BLOCK = 2048


def _transform_body(x_ref, w_ref, b_ref, o_ref):
    o_ref[...] = lax.dot_general(
        x_ref[...], w_ref[...],
        dimension_numbers=(((1,), (1,)), ((), ())),
        preferred_element_type=jnp.float32,
    ) + b_ref[...]


def _transform_table(table, W, b):
    n = table.shape[0]
    grid = n // _ROWS_PER_BLOCK
    return pl.pallas_call(
        _transform_body,
        grid=(grid,),
        in_specs=[
            pl.BlockSpec((_ROWS_PER_BLOCK, EMBED), lambda i: (i, 0)),
            pl.BlockSpec((EMBED, EMBED), lambda i: (0, 0)),
            pl.BlockSpec((1, EMBED), lambda i: (0, 0)),
        ],
        out_specs=pl.BlockSpec((_ROWS_PER_BLOCK, EMBED), lambda i: (i, 0)),
        out_shape=jax.ShapeDtypeStruct((n, EMBED), jnp.float32),
    )(table, W, b.reshape(1, EMBED))


def _time_body(idx_ref, tab_ref, o_ref):
    bsz = idx_ref.shape[-1]
    idx = idx_ref[0]
    iot = lax.broadcasted_iota(jnp.int32, (TIME_V, bsz), 0)
    onehot = (jnp.broadcast_to(idx, (TIME_V, bsz)) == iot).astype(jnp.float32)
    o_ref[0] = lax.dot_general(
        tab_ref[...], onehot,
        dimension_numbers=(((0,), (0,)), ((), ())),
        preferred_element_type=jnp.float32,
    )


def _time_lookup(time_idx_t, time_table):
    hist, bsz = time_idx_t.shape
    return pl.pallas_call(
        _time_body,
        grid=(hist,),
        in_specs=[
            pl.BlockSpec((1, 1, bsz), lambda i: (i, 0, 0)),
            pl.BlockSpec((TIME_V, TIME_D), lambda i: (0, 0)),
        ],
        out_specs=pl.BlockSpec((1, TIME_D, bsz), lambda i: (i, 0, 0)),
        out_shape=jax.ShapeDtypeStruct((hist, TIME_D, bsz), jnp.float32),
    )(time_idx_t.reshape(hist, 1, bsz), time_table)


@functools.lru_cache(maxsize=None)
def _make_sc_gather(B):
    info = plsc.get_sparse_core_info()
    NC, NS = info.num_cores, info.num_subcores
    NW = NC * NS
    rows_per_w = B // NW
    CH = 128
    n_ch = rows_per_w // CH
    NBUF = 5
    n_grp = n_ch // NBUF
    mesh = plsc.VectorSubcoreMesh(core_axis_name="c", subcore_axis_name="s")

    @functools.partial(
        pl.kernel,
        mesh=mesh,
        out_type=jax.ShapeDtypeStruct((B, EMBED), jnp.float32),
        scratch_types=[
            pltpu.VMEM((n_ch, CH), jnp.int32),
            pltpu.VMEM((NBUF, CH, EMBED), jnp.float32),
            pltpu.SemaphoreType.DMA,
            pltpu.SemaphoreType.DMA,
        ],
        compiler_params=pltpu.CompilerParams(use_tc_tiling_on_sc=True),
    )
    def sc_gather(ptab_hbm, pidx_hbm, pout_hbm, pidx_v, prow_v, psem, osem):
        wid = lax.axis_index("s") * NC + lax.axis_index("c")
        pltpu.sync_copy(pidx_hbm.at[wid], pidx_v)

        def _drain_one_writeback(k):
            pltpu.make_async_copy(
                prow_v.at[k], pout_hbm.at[pl.ds(0, CH)], osem).wait()

        def body(g, carry):
            j0 = g * NBUF
            cps = []
            for k in range(NBUF):
                @pl.when(j0 + k >= NBUF)
                def _(k=k):
                    _drain_one_writeback(k)
                cps.append(pltpu.async_copy(
                    ptab_hbm.at[pidx_v.at[j0 + k]], prow_v.at[k], psem))
            for k in range(NBUF):
                cps[k].wait()
                out0 = wid * rows_per_w + (j0 + k) * CH
                pltpu.async_copy(
                    prow_v.at[k], pout_hbm.at[pl.ds(out0, CH)], osem)
            return carry

        lax.fori_loop(0, n_grp, body, 0)
        for k in range(NBUF):
            _drain_one_writeback(k)

    return sc_gather


def kernel(poi_input, time_input, poi_table, time_table, W, b):
    bsz, hist = poi_input.shape
    B = bsz * hist
    t_table = _transform_table(poi_table, W, b)
    time_out = _time_lookup(time_input.T, time_table)
    info = plsc.get_sparse_core_info()
    nw = info.num_cores * info.num_subcores
    pidx = poi_input.T.reshape(nw, B // (nw * 128), 128)
    poi_out = _make_sc_gather(B)(t_table, pidx)
    return (poi_out.reshape(hist, bsz, EMBED).transpose(1, 0, 2),
            time_out.transpose(2, 0, 1))


# speedup vs baseline: 12.4405x; 1.0059x over previous
import functools

import jax
import jax.numpy as jnp
from jax import lax
from jax.experimental import pallas as pl
from jax.experimental.pallas import tpu as pltpu
from jax.experimental.pallas import tpu_sc as plsc

EMBED = 128
TIME_D = 32
TIME_V = 24
_ROWS_PER_BLOCK = 5000
_TIME_BLOCK = 2048


def _transform_body(x_ref, w_ref, b_ref, o_ref):
    o_ref[...] = lax.dot_general(
        x_ref[...], w_ref[...],
        dimension_numbers=(((1,), (1,)), ((), ())),
        preferred_element_type=jnp.float32,
    ) + b_ref[...]


def _transform_table(table, W, b):
    n = table.shape[0]
    grid = n // _ROWS_PER_BLOCK
    return pl.pallas_call(
        _transform_body,
        grid=(grid,),
        in_specs=[
            pl.BlockSpec((_ROWS_PER_BLOCK, EMBED), lambda i: (i, 0)),
            pl.BlockSpec((EMBED, EMBED), lambda i: (0, 0)),
            pl.BlockSpec((1, EMBED), lambda i: (0, 0)),
        ],
        out_specs=pl.BlockSpec((_ROWS_PER_BLOCK, EMBED), lambda i: (i, 0)),
        out_shape=jax.ShapeDtypeStruct((n, EMBED), jnp.float32),
    )(table, W, b.reshape(1, EMBED))


def _time_body(idx_ref, tab_ref, o_ref):
    bsz = idx_ref.shape[-1]
    idx = idx_ref[0]
    iot = lax.broadcasted_iota(jnp.int32, (TIME_V, bsz), 0)
    onehot = (jnp.broadcast_to(idx, (TIME_V, bsz)) == iot).astype(jnp.float32)
    o_ref[0] = lax.dot_general(
        tab_ref[...], onehot,
        dimension_numbers=(((0,), (0,)), ((), ())),
        preferred_element_type=jnp.float32,
    )


def _time_lookup(time_idx_t, time_table):
    hist, bsz = time_idx_t.shape
    return pl.pallas_call(
        _time_body,
        grid=(hist,),
        in_specs=[
            pl.BlockSpec((1, 1, bsz), lambda i: (i, 0, 0)),
            pl.BlockSpec((TIME_V, TIME_D), lambda i: (0, 0)),
        ],
        out_specs=pl.BlockSpec((1, TIME_D, bsz), lambda i: (i, 0, 0)),
        out_shape=jax.ShapeDtypeStruct((hist, TIME_D, bsz), jnp.float32),
    )(time_idx_t.reshape(hist, 1, bsz), time_table)


@functools.lru_cache(maxsize=None)
def _make_sc_gather(B):
    info = plsc.get_sparse_core_info()
    NC, NS = info.num_cores, info.num_subcores
    NW = NC * NS
    rows_per_w = B // NW
    CH = 128
    GPC = 1
    n_ch = rows_per_w // CH
    NBUF = 5
    n_grp = n_ch // (NBUF * GPC)
    mesh = plsc.VectorSubcoreMesh(core_axis_name="c", subcore_axis_name="s")

    @functools.partial(
        pl.kernel,
        mesh=mesh,
        out_type=jax.ShapeDtypeStruct((B, EMBED), jnp.float32),
        scratch_types=[
            pltpu.VMEM((n_ch, CH), jnp.int32),
            pltpu.VMEM((NBUF, GPC * CH, EMBED), jnp.float32),
            pltpu.SemaphoreType.DMA,
            pltpu.SemaphoreType.DMA,
        ],
        compiler_params=pltpu.CompilerParams(use_tc_tiling_on_sc=True),
    )
    def sc_gather(ptab_hbm, pidx_hbm, pout_hbm, pidx_v, prow_v, psem, osem):
        wid = lax.axis_index("s") * NC + lax.axis_index("c")
        pltpu.sync_copy(pidx_hbm.at[wid], pidx_v)

        def _drain_one_writeback(k):
            pltpu.make_async_copy(
                prow_v.at[k], pout_hbm.at[pl.ds(0, GPC * CH)], osem).wait()

        def body(g, carry):
            j0 = g * NBUF * GPC
            cps = []
            for k in range(NBUF):
                @pl.when(j0 + k * GPC >= NBUF * GPC)
                def _(k=k):
                    _drain_one_writeback(k)
                for q in range(GPC):
                    cps.append(pltpu.async_copy(
                        ptab_hbm.at[pidx_v.at[j0 + k * GPC + q]],
                        prow_v.at[k, pl.ds(q * CH, CH)], psem))
            for k in range(NBUF):
                for q in range(GPC):
                    cps[k * GPC + q].wait()
                out0 = wid * rows_per_w + (j0 + k * GPC) * CH
                pltpu.async_copy(
                    prow_v.at[k], pout_hbm.at[pl.ds(out0, GPC * CH)], osem)
            return carry

        lax.fori_loop(0, n_grp, body, 0)
        for k in range(NBUF):
            _drain_one_writeback(k)

    return sc_gather


def kernel(poi_input, time_input, poi_table, time_table, W, b):
    bsz, hist = poi_input.shape
    B = bsz * hist
    t_table = _transform_table(poi_table, W, b)
    time_out = _time_lookup(time_input.T, time_table)
    info = plsc.get_sparse_core_info()
    nw = info.num_cores * info.num_subcores
    pidx = poi_input.T.reshape(nw, B // (nw * 128), 128)
    poi_out = _make_sc_gather(B)(t_table, pidx)
    return (poi_out.reshape(hist, bsz, EMBED).transpose(1, 0, 2),
            time_out.transpose(2, 0, 1))


# speedup vs baseline: 12.9405x; 1.0402x over previous
import functools

import jax
import jax.numpy as jnp
from jax import lax
from jax.experimental import pallas as pl
from jax.experimental.pallas import tpu as pltpu
from jax.experimental.pallas import tpu_sc as plsc

EMBED = 128
TIME_D = 32
TIME_V = 24
_ROWS_PER_BLOCK = 10000
_TIME_BLOCK = 2048


def _transform_body(x_ref, w_ref, b_ref, o_ref):
    o_ref[...] = lax.dot_general(
        x_ref[...], w_ref[...],
        dimension_numbers=(((1,), (1,)), ((), ())),
        preferred_element_type=jnp.float32,
    ) + b_ref[...]


def _transform_table(table, W, b):
    n = table.shape[0]
    grid = n // _ROWS_PER_BLOCK
    return pl.pallas_call(
        _transform_body,
        grid=(grid,),
        in_specs=[
            pl.BlockSpec((_ROWS_PER_BLOCK, EMBED), lambda i: (i, 0)),
            pl.BlockSpec((EMBED, EMBED), lambda i: (0, 0)),
            pl.BlockSpec((1, EMBED), lambda i: (0, 0)),
        ],
        out_specs=pl.BlockSpec((_ROWS_PER_BLOCK, EMBED), lambda i: (i, 0)),
        out_shape=jax.ShapeDtypeStruct((n, EMBED), jnp.float32),
    )(table, W, b.reshape(1, EMBED))


def _time_body(idx_ref, tab_ref, o_ref):
    bsz = idx_ref.shape[-1]
    idx = idx_ref[0]
    iot = lax.broadcasted_iota(jnp.int32, (TIME_V, bsz), 0)
    onehot = (jnp.broadcast_to(idx, (TIME_V, bsz)) == iot).astype(jnp.float32)
    o_ref[0] = lax.dot_general(
        tab_ref[...], onehot,
        dimension_numbers=(((0,), (0,)), ((), ())),
        preferred_element_type=jnp.float32,
    )


def _time_lookup(time_idx_t, time_table):
    hist, bsz = time_idx_t.shape
    return pl.pallas_call(
        _time_body,
        grid=(hist,),
        in_specs=[
            pl.BlockSpec((1, 1, bsz), lambda i: (i, 0, 0)),
            pl.BlockSpec((TIME_V, TIME_D), lambda i: (0, 0)),
        ],
        out_specs=pl.BlockSpec((1, TIME_D, bsz), lambda i: (i, 0, 0)),
        out_shape=jax.ShapeDtypeStruct((hist, TIME_D, bsz), jnp.float32),
    )(time_idx_t.reshape(hist, 1, bsz), time_table)


@functools.lru_cache(maxsize=None)
def _make_sc_gather(B):
    info = plsc.get_sparse_core_info()
    NC, NS = info.num_cores, info.num_subcores
    NW = NC * NS
    rows_per_w = B // NW
    CH = 128
    GPC = 1
    n_ch = rows_per_w // CH
    NBUF = 5
    n_grp = n_ch // (NBUF * GPC)
    mesh = plsc.VectorSubcoreMesh(core_axis_name="c", subcore_axis_name="s")

    @functools.partial(
        pl.kernel,
        mesh=mesh,
        out_type=jax.ShapeDtypeStruct((B, EMBED), jnp.float32),
        scratch_types=[
            pltpu.VMEM((n_ch, CH), jnp.int32),
            pltpu.VMEM((NBUF, GPC * CH, EMBED), jnp.float32),
            pltpu.SemaphoreType.DMA,
            pltpu.SemaphoreType.DMA,
        ],
        compiler_params=pltpu.CompilerParams(use_tc_tiling_on_sc=True),
    )
    def sc_gather(ptab_hbm, pidx_hbm, pout_hbm, pidx_v, prow_v, psem, osem):
        wid = lax.axis_index("s") * NC + lax.axis_index("c")
        pltpu.sync_copy(pidx_hbm.at[wid], pidx_v)

        def _drain_one_writeback(k):
            pltpu.make_async_copy(
                prow_v.at[k], pout_hbm.at[pl.ds(0, GPC * CH)], osem).wait()

        def body(g, carry):
            j0 = g * NBUF * GPC
            cps = []
            for k in range(NBUF):
                @pl.when(j0 + k * GPC >= NBUF * GPC)
                def _(k=k):
                    _drain_one_writeback(k)
                for q in range(GPC):
                    cps.append(pltpu.async_copy(
                        ptab_hbm.at[pidx_v.at[j0 + k * GPC + q]],
                        prow_v.at[k, pl.ds(q * CH, CH)], psem))
            for k in range(NBUF):
                for q in range(GPC):
                    cps[k * GPC + q].wait()
                out0 = wid * rows_per_w + (j0 + k * GPC) * CH
                pltpu.async_copy(
                    prow_v.at[k], pout_hbm.at[pl.ds(out0, GPC * CH)], osem)
            return carry

        lax.fori_loop(0, n_grp, body, 0)
        for k in range(NBUF):
            _drain_one_writeback(k)

    return sc_gather


def kernel(poi_input, time_input, poi_table, time_table, W, b):
    bsz, hist = poi_input.shape
    B = bsz * hist
    t_table = _transform_table(poi_table, W, b)
    time_out = _time_lookup(time_input.T, time_table)
    info = plsc.get_sparse_core_info()
    nw = info.num_cores * info.num_subcores
    pidx = poi_input.T.reshape(nw, B // (nw * 128), 128)
    poi_out = _make_sc_gather(B)(t_table, pidx)
    return (poi_out.reshape(hist, bsz, EMBED).transpose(1, 0, 2),
            time_out.transpose(2, 0, 1))
